# Initial kernel scaffold; baseline (speedup 1.0000x reference)
import jax
import jax.numpy as jnp
from jax.experimental import pallas as pl


def kernel(x, edge_index, edge_attr, spatial_coords, gat_W, gat_att_src, gat_att_dst, gat_bias, sa_w1, sa_b1, sa_w2, sa_b2, ft_w, ft_b, bn_gamma, bn_beta):
    raise NotImplementedError("write your pallas kernel here")


# speedup vs baseline: 46.9831x; 46.9831x over previous
import jax
import jax.numpy as jnp
from jax import lax
from jax.experimental import pallas as pl
from jax.experimental.pallas import tpu as pltpu
from jax.experimental.pallas import tpu_sc as plsc

N = 10000
D = 128
H = 4
C = 32
NEG = 0.2
EPS = 1e-5

NC, NS, L = 2, 16, 16
DH = D // NC
HC = H // NC
W = DH + 8
CH = 128
NP = 10112
RPT = NP // NS


def _dense_pre(x_ref, w_ref, a8_ref, fw_ref, fb_ref, g_ref, b_ref,
               xw_ref, asd_ref, t_ref, m_ref):
    x = x_ref[...]
    xw = jnp.dot(x, w_ref[...], preferred_element_type=jnp.float32)
    asd = jnp.dot(xw, a8_ref[...], preferred_element_type=jnp.float32)
    xw_ref[...] = xw
    asd_ref[...] = asd
    mraw = jnp.max(asd[:, :H]) + jnp.max(asd[:, H:])
    m = jnp.maximum(mraw, NEG * mraw)
    m_ref[...] = jnp.broadcast_to(m, (1, D))
    y = jnp.dot(x, fw_ref[...], preferred_element_type=jnp.float32) + fb_ref[...]
    mu = jnp.mean(y, axis=0, keepdims=True)
    var = jnp.mean((y - mu) ** 2, axis=0, keepdims=True)
    t = (y - mu) / jnp.sqrt(var + EPS) * g_ref[...] + b_ref[...]
    t_ref[...] = jnp.maximum(t, 0.0)


def _sc_edge(src_ref, dst_ref, asd_ref, xw2_ref, m_ref, raw_ref,
             msg_sh, asd_sh, rows_v, msg_v, p_v,
             src_v, dst_v, sidx_v, fsb_v, fdb_v, ga_v, gb_v,
             m_v, stage_v, sem):
    cid = lax.axis_index("c")
    sid = lax.axis_index("s")
    cpw = src_ref.shape[0] // NS // CH

    pltpu.sync_copy(m_ref, m_v)

    spt = asd_ref.shape[0] // NS
    s0 = pl.multiple_of(sid * spt, 8)
    for q in range(spt // 1264):
        pltpu.sync_copy(asd_ref.at[pl.ds(s0 + q * 1264, 1264)], stage_v)
        pltpu.sync_copy(stage_v, asd_sh.at[pl.ds(s0 + q * 1264, 1264)])

    zv = jnp.zeros((L,), jnp.float32)

    @pl.loop(0, CH)
    def _zero_msg(k):
        for o in (0, L, 2 * L, 3 * L, W - L):
            msg_v[k, pl.ds(o, L)] = zv

    r0 = pl.multiple_of(sid * RPT, 8)
    for off, sz in ((0, 128), (128, 128), (256, 128), (384, 128), (512, 120)):
        pltpu.sync_copy(msg_v.at[pl.ds(0, sz)], msg_sh.at[pl.ds(r0 + off, sz)])
    plsc.subcore_barrier()

    iota = lax.iota(jnp.int32, L)
    mvec = m_v[...]
    e_base = sid * cpw * CH

    @pl.loop(0, cpw)
    def _chunk(t):
        e0 = pl.multiple_of(e_base + t * CH, 8)
        pltpu.sync_copy(src_ref.at[pl.ds(e0, CH)], src_v)
        pltpu.sync_copy(dst_ref.at[pl.ds(e0, CH)], dst_v)

        for k0 in range(0, CH, L):
            s16 = src_v[pl.ds(k0, L)]
            d16 = dst_v[pl.ds(k0, L)]
            sidx_v[pl.ds(k0, L)] = s16 * 2 + cid
            fsb_v[0, pl.ds(k0, L)] = s16 * 8 + cid * HC
            fsb_v[1, pl.ds(k0, L)] = s16 * 8 + (cid * HC + 1)
            fdb_v[0, pl.ds(k0, L)] = d16 * 8 + (H + cid * HC)
            fdb_v[1, pl.ds(k0, L)] = d16 * 8 + (H + cid * HC + 1)
        gxw = pltpu.async_copy(xw2_ref.at[sidx_v], rows_v, sem)
        for j in range(HC):
            pltpu.sync_copy(asd_sh.at[fsb_v.at[j]], ga_v.at[j])
            pltpu.sync_copy(asd_sh.at[fdb_v.at[j]], gb_v.at[j])
        gxw.wait()

        for k0 in range(0, CH, L):
            kvec = iota + k0
            for j in range(HC):
                al = ga_v[j, pl.ds(k0, L)] + gb_v[j, pl.ds(k0, L)]
                al = jnp.maximum(al, NEG * al)
                pv = jnp.exp(al - mvec)
                p_v[j, pl.ds(k0, L)] = pv
                plsc.store_scatter(
                    msg_v, [kvec, jnp.full((L,), DH + j, jnp.int32)], pv)

        @pl.loop(0, CH, step=L)
        def _row(k0):
            for j in range(HC):
                ph = p_v[j, pl.ds(k0, L)]
                for k in range(L):
                    coef = ph[k]
                    for i in range(2):
                        sl = pl.ds(j * C + i * L, L)
                        msg_v[k0 + k, sl] = rows_v[k0 + k, sl] * coef

        pltpu.sync_copy(msg_v, msg_sh.at[dst_v], add=True)

    plsc.subcore_barrier()
    for off, sz in ((0, 128), (128, 128), (256, 128), (384, 128), (512, 120)):
        pltpu.sync_copy(msg_sh.at[pl.ds(r0 + off, sz)],
                        raw_ref.at[cid, pl.ds(r0 + off, sz)])


def _dense_post(raw_ref, a0_ref, asd_ref, xw_ref, t_ref, exp_ref,
                bias_ref, m_ref, o_ref):
    num = raw_ref[:N, :]
    asum4 = a0_ref[...]
    asd = asd_ref[...]
    aself = asd[:, :H] + asd[:, H:]
    aself = jnp.maximum(aself, NEG * aself)
    es4 = jnp.exp(aself - m_ref[0:1, 0:1])
    expm = exp_ref[...]
    den = jnp.dot(asum4 + es4, expm, preferred_element_type=jnp.float32) + 1e-16
    num = num + jnp.dot(es4, expm, preferred_element_type=jnp.float32) * xw_ref[...]
    o_ref[...] = num / den + bias_ref[...] + t_ref[...]


def kernel(x, edge_index, edge_attr, spatial_coords,
           gat_W, gat_att_src, gat_att_dst, gat_bias,
           sa_w1, sa_b1, sa_w2, sa_b2,
           ft_w, ft_b, bn_gamma, bn_beta):
    f32 = jnp.float32
    expm = jnp.kron(jnp.eye(H, dtype=f32), jnp.ones((1, C), f32))
    a8 = jnp.concatenate(
        [expm.T * gat_att_src.reshape(-1)[:, None],
         expm.T * gat_att_dst.reshape(-1)[:, None]], axis=1)

    xw, asd, t, m = pl.pallas_call(
        _dense_pre,
        out_shape=[
            jax.ShapeDtypeStruct((N, D), f32),
            jax.ShapeDtypeStruct((N, 2 * H), f32),
            jax.ShapeDtypeStruct((N, D), f32),
            jax.ShapeDtypeStruct((1, D), f32),
        ],
    )(x, gat_W, a8, ft_w, ft_b.reshape(1, D), bn_gamma.reshape(1, D),
      bn_beta.reshape(1, D))

    e = edge_index.shape[1]
    cpw = -(-e // (NS * CH))
    e_pad = cpw * NS * CH
    srcp = jnp.concatenate([edge_index[0], jnp.zeros((e_pad - e,), jnp.int32)])
    dstp = jnp.concatenate([edge_index[1], jnp.full((e_pad - e,), N, jnp.int32)])
    asd_flat = jnp.concatenate(
        [asd, jnp.zeros((NP - N, 2 * H), f32)]).reshape(-1)
    xw2 = xw.reshape(N * NC, DH)
    m16 = m[0, :L]

    raw = pl.kernel(
        _sc_edge,
        out_type=jax.ShapeDtypeStruct((NC, NP, W), f32),
        mesh=plsc.VectorSubcoreMesh(core_axis_name="c", subcore_axis_name="s",
                                    num_cores=NC),
        compiler_params=pltpu.CompilerParams(needs_layout_passes=False,
                                             use_tc_tiling_on_sc=False),
        scratch_types=[
            pltpu.VMEM_SHARED((NP, W), f32),
            pltpu.VMEM_SHARED((NP * 2 * H,), f32),
            pltpu.VMEM((CH, DH), f32),
            pltpu.VMEM((CH, W), f32),
            pltpu.VMEM((HC, CH), f32),
            pltpu.VMEM((CH,), jnp.int32),
            pltpu.VMEM((CH,), jnp.int32),
            pltpu.VMEM((CH,), jnp.int32),
            pltpu.VMEM((HC, CH), jnp.int32),
            pltpu.VMEM((HC, CH), jnp.int32),
            pltpu.VMEM((HC, CH), f32),
            pltpu.VMEM((HC, CH), f32),
            pltpu.VMEM((L,), f32),
            pltpu.VMEM((1264,), f32),
            pltpu.SemaphoreType.DMA,
        ],
    )(srcp, dstp, asd_flat, xw2, m16)

    num = jnp.concatenate([raw[0, :, :DH], raw[1, :, :DH]], axis=1)
    a0 = jnp.concatenate(
        [raw[0, :N, DH:DH + HC], raw[1, :N, DH:DH + HC]], axis=1)
    out = pl.pallas_call(
        _dense_post,
        out_shape=jax.ShapeDtypeStruct((N, D), f32),
    )(num, a0, asd, xw, t, expm, gat_bias.reshape(1, D), m)
    return out


# speedup vs baseline: 47.9992x; 1.0216x over previous
import jax
import jax.numpy as jnp
from jax import lax
from jax.experimental import pallas as pl
from jax.experimental.pallas import tpu as pltpu
from jax.experimental.pallas import tpu_sc as plsc

N = 10000
D = 128
H = 4
C = 32
NEG = 0.2
EPS = 1e-5

NC, NS, L = 2, 16, 16
DH = D // NC
HC = H // NC
W = DH + 8
CH = 128
SCB = 4
NP = 10112
RPT = NP // NS


def _dense_pre(x_ref, w_ref, a8_ref, fw_ref, fb_ref, g_ref, b_ref,
               xw_ref, asd_ref, t_ref, m_ref):
    x = x_ref[...]
    xw = jnp.dot(x, w_ref[...], preferred_element_type=jnp.float32)
    asd = jnp.dot(xw, a8_ref[...], preferred_element_type=jnp.float32)
    xw_ref[...] = xw
    asd_ref[...] = asd
    mraw = jnp.max(asd[:, :H]) + jnp.max(asd[:, H:])
    m = jnp.maximum(mraw, NEG * mraw)
    m_ref[...] = jnp.broadcast_to(m, (1, D))
    y = jnp.dot(x, fw_ref[...], preferred_element_type=jnp.float32) + fb_ref[...]
    mu = jnp.mean(y, axis=0, keepdims=True)
    var = jnp.mean((y - mu) ** 2, axis=0, keepdims=True)
    t = (y - mu) / jnp.sqrt(var + EPS) * g_ref[...] + b_ref[...]
    t_ref[...] = jnp.maximum(t, 0.0)


def _sc_edge(src_ref, dst_ref, asd_ref, xw2_ref, m_ref, raw_ref,
             msg_sh, asd_sh, rows_v, msg_v, p_v,
             src_v, dst_v, dstx_v, sidx_v, fsb_v, fdb_v, ga_v, gb_v,
             m_v, stage_v, semg, seml, sems):
    cid = lax.axis_index("c")
    sid = lax.axis_index("s")
    nsc = src_ref.shape[0] // NS // (SCB * CH)

    pltpu.sync_copy(m_ref, m_v)

    spt = asd_ref.shape[0] // NS
    s0 = pl.multiple_of(sid * spt, 8)
    for q in range(spt // 1264):
        pltpu.sync_copy(asd_ref.at[pl.ds(s0 + q * 1264, 1264)], stage_v)
        pltpu.sync_copy(stage_v, asd_sh.at[pl.ds(s0 + q * 1264, 1264)])

    zv = jnp.zeros((L,), jnp.float32)

    @pl.loop(0, CH)
    def _zero_msg(k):
        for b in range(SCB):
            for o in (0, L, 2 * L, 3 * L, W - L):
                msg_v[b, k, pl.ds(o, L)] = zv

    r0 = pl.multiple_of(sid * RPT, 8)
    for off, sz in ((0, 128), (128, 128), (256, 128), (384, 128), (512, 120)):
        pltpu.sync_copy(msg_v.at[0, pl.ds(0, sz)],
                        msg_sh.at[pl.ds(r0 + off, sz)])
    plsc.subcore_barrier()

    iota = lax.iota(jnp.int32, L)
    mvec = m_v[...]
    e_base = sid * nsc * SCB * CH

    @pl.loop(0, nsc)
    def _schunk(t):
        e0 = pl.multiple_of(e_base + t * SCB * CH, 8)
        pltpu.sync_copy(src_ref.at[pl.ds(e0, SCB * CH)], src_v)
        pltpu.sync_copy(dst_ref.at[pl.ds(e0, SCB * CH)], dst_v)

        for b in range(SCB):
            for k0 in range(0, CH, L):
                kk = b * CH + k0
                s16 = src_v[pl.ds(kk, L)]
                d16 = dst_v[pl.ds(kk, L)]
                sidx_v[pl.ds(kk, L)] = s16 * 2 + cid
                dstx_v[b, pl.ds(k0, L)] = d16
                fsb_v[b, 0, pl.ds(k0, L)] = s16 * 8 + cid * HC
                fsb_v[b, 1, pl.ds(k0, L)] = s16 * 8 + (cid * HC + 1)
                fdb_v[b, 0, pl.ds(k0, L)] = d16 * 8 + (H + cid * HC)
                fdb_v[b, 1, pl.ds(k0, L)] = d16 * 8 + (H + cid * HC + 1)

        gx = []
        for b in range(SCB):
            gx.append(pltpu.async_copy(
                xw2_ref.at[sidx_v.at[pl.ds(b * CH, CH)]], rows_v.at[b], semg))
        gl = []
        for b in range(SCB):
            for j in range(HC):
                gl.append(pltpu.async_copy(
                    asd_sh.at[fsb_v.at[b, j]], ga_v.at[b, j], seml))
                gl.append(pltpu.async_copy(
                    asd_sh.at[fdb_v.at[b, j]], gb_v.at[b, j], seml))
        for g in gl:
            g.wait()
        for g in gx:
            g.wait()

        sc = []
        for b in range(SCB):
            for k0 in range(0, CH, L):
                kvec = iota + k0
                for j in range(HC):
                    al = ga_v[b, j, pl.ds(k0, L)] + gb_v[b, j, pl.ds(k0, L)]
                    al = jnp.maximum(al, NEG * al)
                    pv = jnp.exp(al - mvec)
                    p_v[j, pl.ds(k0, L)] = pv
                    plsc.store_scatter(
                        msg_v.at[b], [kvec, jnp.full((L,), DH + j, jnp.int32)],
                        pv)

            @pl.loop(0, CH, step=L)
            def _row(k0):
                for j in range(HC):
                    ph = p_v[j, pl.ds(k0, L)]
                    for k in range(L):
                        coef = ph[k]
                        for i in range(2):
                            sl = pl.ds(j * C + i * L, L)
                            msg_v[b, k0 + k, sl] = rows_v[b, k0 + k, sl] * coef

            sc.append(pltpu.async_copy(
                msg_v.at[b], msg_sh.at[dstx_v.at[b]], sems, add=True))
        for g in sc:
            g.wait()

    plsc.subcore_barrier()
    for off, sz in ((0, 128), (128, 128), (256, 128), (384, 128), (512, 120)):
        pltpu.sync_copy(msg_sh.at[pl.ds(r0 + off, sz)],
                        raw_ref.at[cid, pl.ds(r0 + off, sz)])


def _dense_post(raw_ref, a0_ref, asd_ref, xw_ref, t_ref, exp_ref,
                bias_ref, m_ref, o_ref):
    num = raw_ref[:N, :]
    asum4 = a0_ref[...]
    asd = asd_ref[...]
    aself = asd[:, :H] + asd[:, H:]
    aself = jnp.maximum(aself, NEG * aself)
    es4 = jnp.exp(aself - m_ref[0:1, 0:1])
    expm = exp_ref[...]
    den = jnp.dot(asum4 + es4, expm, preferred_element_type=jnp.float32) + 1e-16
    num = num + jnp.dot(es4, expm, preferred_element_type=jnp.float32) * xw_ref[...]
    o_ref[...] = num / den + bias_ref[...] + t_ref[...]


def kernel(x, edge_index, edge_attr, spatial_coords,
           gat_W, gat_att_src, gat_att_dst, gat_bias,
           sa_w1, sa_b1, sa_w2, sa_b2,
           ft_w, ft_b, bn_gamma, bn_beta):
    f32 = jnp.float32
    expm = jnp.kron(jnp.eye(H, dtype=f32), jnp.ones((1, C), f32))
    a8 = jnp.concatenate(
        [expm.T * gat_att_src.reshape(-1)[:, None],
         expm.T * gat_att_dst.reshape(-1)[:, None]], axis=1)

    xw, asd, t, m = pl.pallas_call(
        _dense_pre,
        out_shape=[
            jax.ShapeDtypeStruct((N, D), f32),
            jax.ShapeDtypeStruct((N, 2 * H), f32),
            jax.ShapeDtypeStruct((N, D), f32),
            jax.ShapeDtypeStruct((1, D), f32),
        ],
    )(x, gat_W, a8, ft_w, ft_b.reshape(1, D), bn_gamma.reshape(1, D),
      bn_beta.reshape(1, D))

    e = edge_index.shape[1]
    nsc = -(-e // (NS * SCB * CH))
    e_pad = nsc * NS * SCB * CH
    srcp = jnp.concatenate([edge_index[0], jnp.zeros((e_pad - e,), jnp.int32)])
    dstp = jnp.concatenate([edge_index[1], jnp.full((e_pad - e,), N, jnp.int32)])
    asd_flat = jnp.concatenate(
        [asd, jnp.zeros((NP - N, 2 * H), f32)]).reshape(-1)
    xw2 = xw.reshape(N * NC, DH)
    m16 = m[0, :L]

    raw = pl.kernel(
        _sc_edge,
        out_type=jax.ShapeDtypeStruct((NC, NP, W), f32),
        mesh=plsc.VectorSubcoreMesh(core_axis_name="c", subcore_axis_name="s",
                                    num_cores=NC),
        compiler_params=pltpu.CompilerParams(needs_layout_passes=False,
                                             use_tc_tiling_on_sc=False),
        scratch_types=[
            pltpu.VMEM_SHARED((NP, W), f32),
            pltpu.VMEM_SHARED((NP * 2 * H,), f32),
            pltpu.VMEM((SCB, CH, DH), f32),
            pltpu.VMEM((SCB, CH, W), f32),
            pltpu.VMEM((HC, CH), f32),
            pltpu.VMEM((SCB * CH,), jnp.int32),
            pltpu.VMEM((SCB * CH,), jnp.int32),
            pltpu.VMEM((SCB, CH), jnp.int32),
            pltpu.VMEM((SCB * CH,), jnp.int32),
            pltpu.VMEM((SCB, HC, CH), jnp.int32),
            pltpu.VMEM((SCB, HC, CH), jnp.int32),
            pltpu.VMEM((SCB, HC, CH), f32),
            pltpu.VMEM((SCB, HC, CH), f32),
            pltpu.VMEM((L,), f32),
            pltpu.VMEM((1264,), f32),
            pltpu.SemaphoreType.DMA,
            pltpu.SemaphoreType.DMA,
            pltpu.SemaphoreType.DMA,
        ],
    )(srcp, dstp, asd_flat, xw2, m16)

    num = jnp.concatenate([raw[0, :, :DH], raw[1, :, :DH]], axis=1)
    a0 = jnp.concatenate(
        [raw[0, :N, DH:DH + HC], raw[1, :N, DH:DH + HC]], axis=1)
    out = pl.pallas_call(
        _dense_post,
        out_shape=jax.ShapeDtypeStruct((N, D), f32),
    )(num, a0, asd, xw, t, expm, gat_bias.reshape(1, D), m)
    return out


# speedup vs baseline: 54.4606x; 1.1346x over previous
import jax
import jax.numpy as jnp
from jax import lax
from jax.experimental import pallas as pl
from jax.experimental.pallas import tpu as pltpu
from jax.experimental.pallas import tpu_sc as plsc

N = 10000
D = 128
H = 4
C = 32
NEG = 0.2
EPS = 1e-5

NC, NS, L = 2, 16, 16
DH = D // NC
HC = H // NC
W = DH + 8
CH = 128
SCB = 2
EC = SCB * CH
NP = 10112
RPT = NP // NS


def _dense_pre(x_ref, w_ref, a8_ref, fw_ref, fb_ref, g_ref, b_ref,
               xw_ref, asd_ref, t_ref, m_ref):
    x = x_ref[...]
    xw = jnp.dot(x, w_ref[...], preferred_element_type=jnp.float32)
    asd = jnp.dot(xw, a8_ref[...], preferred_element_type=jnp.float32)
    xw_ref[...] = xw
    asd_ref[...] = asd
    mraw = jnp.max(asd[:, :H]) + jnp.max(asd[:, H:])
    m = jnp.maximum(mraw, NEG * mraw)
    m_ref[...] = jnp.broadcast_to(m, (1, D))
    y = jnp.dot(x, fw_ref[...], preferred_element_type=jnp.float32) + fb_ref[...]
    mu = jnp.mean(y, axis=0, keepdims=True)
    var = jnp.mean((y - mu) ** 2, axis=0, keepdims=True)
    t = (y - mu) / jnp.sqrt(var + EPS) * g_ref[...] + b_ref[...]
    t_ref[...] = jnp.maximum(t, 0.0)


def _sc_edge(src_ref, dst_ref, asd_ref, xw2_ref, m_ref, raw_ref,
             msg_sh, asd_sh, rows_v, msg_v, p_v,
             src_v, dst_v, dstx_v, sidx_v, fsb_v, fdb_v, ga_v, gb_v,
             m_v, stage_v, semg0, semg1, seml0, seml1, sems0, sems1):
    cid = lax.axis_index("c")
    sid = lax.axis_index("s")
    nsc = src_ref.shape[0] // NS // EC
    semg = (semg0, semg1)
    seml = (seml0, seml1)
    sems = (sems0, sems1)

    pltpu.sync_copy(m_ref, m_v)

    spt = asd_ref.shape[0] // NS
    s0 = pl.multiple_of(sid * spt, 8)
    for q in range(spt // 1264):
        pltpu.sync_copy(asd_ref.at[pl.ds(s0 + q * 1264, 1264)], stage_v)
        pltpu.sync_copy(stage_v, asd_sh.at[pl.ds(s0 + q * 1264, 1264)])

    zv = jnp.zeros((L,), jnp.float32)

    @pl.loop(0, CH)
    def _zero_msg(k):
        for s in range(2):
            for b in range(SCB):
                for o in (0, L, 2 * L, 3 * L, W - L):
                    msg_v[s, b, k, pl.ds(o, L)] = zv

    r0 = pl.multiple_of(sid * RPT, 8)
    for off, sz in ((0, 128), (128, 128), (256, 128), (384, 128), (512, 120)):
        pltpu.sync_copy(msg_v.at[0, 0, pl.ds(0, sz)],
                        msg_sh.at[pl.ds(r0 + off, sz)])
    plsc.subcore_barrier()

    iota = lax.iota(jnp.int32, L)
    fullk = [jnp.full((L,), k, jnp.int32) for k in range(L)]
    colc = [jnp.full((L,), DH + j, jnp.int32) for j in range(HC)]
    mvec = m_v[...]
    e_base = sid * nsc * EC

    def build_fire(s, t):
        e0 = pl.multiple_of(e_base + t * EC, 8)
        pltpu.sync_copy(src_ref.at[pl.ds(e0, EC)], src_v)
        pltpu.sync_copy(dst_ref.at[pl.ds(e0, EC)], dst_v)
        for b in range(SCB):
            for k0 in range(0, CH, L):
                kk = b * CH + k0
                s16 = src_v[pl.ds(kk, L)]
                d16 = dst_v[pl.ds(kk, L)]
                sidx_v[s, pl.ds(kk, L)] = s16 * 2 + cid
                dstx_v[s, b, pl.ds(k0, L)] = d16
                fsb_v[s, b, 0, pl.ds(k0, L)] = s16 * 8 + cid * HC
                fsb_v[s, b, 1, pl.ds(k0, L)] = s16 * 8 + (cid * HC + 1)
                fdb_v[s, b, 0, pl.ds(k0, L)] = d16 * 8 + (H + cid * HC)
                fdb_v[s, b, 1, pl.ds(k0, L)] = d16 * 8 + (H + cid * HC + 1)
        for b in range(SCB):
            pltpu.async_copy(xw2_ref.at[sidx_v.at[s, pl.ds(b * CH, CH)]],
                             rows_v.at[s, b], semg[s])
            for j in range(HC):
                pltpu.async_copy(asd_sh.at[fsb_v.at[s, b, j]],
                                 ga_v.at[s, b, j], seml[s])
                pltpu.async_copy(asd_sh.at[fdb_v.at[s, b, j]],
                                 gb_v.at[s, b, j], seml[s])

    def wait_gathers(s):
        for b in range(SCB):
            pltpu.make_async_copy(
                xw2_ref.at[sidx_v.at[s, pl.ds(b * CH, CH)]],
                rows_v.at[s, b], semg[s]).wait()
            for j in range(HC):
                pltpu.make_async_copy(asd_sh.at[fsb_v.at[s, b, j]],
                                      ga_v.at[s, b, j], seml[s]).wait()
                pltpu.make_async_copy(asd_sh.at[fdb_v.at[s, b, j]],
                                      gb_v.at[s, b, j], seml[s]).wait()

    def compute_fire(s):
        for b in range(SCB):
            for k0 in range(0, CH, L):
                kvec = iota + k0
                for j in range(HC):
                    al = ga_v[s, b, j, pl.ds(k0, L)] + gb_v[s, b, j, pl.ds(k0, L)]
                    al = jnp.maximum(al, NEG * al)
                    pv = jnp.exp(al - mvec)
                    p_v[j, pl.ds(k0, L)] = pv
                    plsc.store_scatter(msg_v.at[s, b], [kvec, colc[j]], pv)

            @pl.loop(0, CH, step=L)
            def _row(k0):
                for j in range(HC):
                    ph = p_v[j, pl.ds(k0, L)]
                    for k in range(L):
                        cf = ph[k]
                        for i in range(2):
                            sl = pl.ds(j * C + i * L, L)
                            msg_v[s, b, k0 + k, sl] = \
                                rows_v[s, b, k0 + k, sl] * cf

            pltpu.async_copy(msg_v.at[s, b], msg_sh.at[dstx_v.at[s, b]],
                             sems[s], add=True)

    def drain_scatter(s):
        for b in range(SCB):
            pltpu.make_async_copy(msg_v.at[s, b], msg_sh.at[dstx_v.at[s, b]],
                                  sems[s]).wait()

    build_fire(0, 0)

    @pl.loop(0, nsc, step=2)
    def _pair(t):
        build_fire(1, t + 1)
        wait_gathers(0)
        compute_fire(0)
        wait_gathers(1)
        compute_fire(1)
        drain_scatter(0)

        @pl.when(t + 2 < nsc)
        def _pf():
            build_fire(0, t + 2)

        drain_scatter(1)

    plsc.subcore_barrier()
    for off, sz in ((0, 128), (128, 128), (256, 128), (384, 128), (512, 120)):
        pltpu.sync_copy(msg_sh.at[pl.ds(r0 + off, sz)],
                        raw_ref.at[cid, pl.ds(r0 + off, sz)])


def _dense_post(raw_ref, a0_ref, asd_ref, xw_ref, t_ref, exp_ref,
                bias_ref, m_ref, o_ref):
    num = raw_ref[:N, :]
    asum4 = a0_ref[...]
    asd = asd_ref[...]
    aself = asd[:, :H] + asd[:, H:]
    aself = jnp.maximum(aself, NEG * aself)
    es4 = jnp.exp(aself - m_ref[0:1, 0:1])
    expm = exp_ref[...]
    den = jnp.dot(asum4 + es4, expm, preferred_element_type=jnp.float32) + 1e-16
    num = num + jnp.dot(es4, expm, preferred_element_type=jnp.float32) * xw_ref[...]
    o_ref[...] = num / den + bias_ref[...] + t_ref[...]


def kernel(x, edge_index, edge_attr, spatial_coords,
           gat_W, gat_att_src, gat_att_dst, gat_bias,
           sa_w1, sa_b1, sa_w2, sa_b2,
           ft_w, ft_b, bn_gamma, bn_beta):
    f32 = jnp.float32
    expm = jnp.kron(jnp.eye(H, dtype=f32), jnp.ones((1, C), f32))
    a8 = jnp.concatenate(
        [expm.T * gat_att_src.reshape(-1)[:, None],
         expm.T * gat_att_dst.reshape(-1)[:, None]], axis=1)

    xw, asd, t, m = pl.pallas_call(
        _dense_pre,
        out_shape=[
            jax.ShapeDtypeStruct((N, D), f32),
            jax.ShapeDtypeStruct((N, 2 * H), f32),
            jax.ShapeDtypeStruct((N, D), f32),
            jax.ShapeDtypeStruct((1, D), f32),
        ],
    )(x, gat_W, a8, ft_w, ft_b.reshape(1, D), bn_gamma.reshape(1, D),
      bn_beta.reshape(1, D))

    e = edge_index.shape[1]
    nsc = -(-e // (NS * EC))
    nsc = nsc + (nsc % 2)
    e_pad = nsc * NS * EC
    srcp = jnp.concatenate([edge_index[0], jnp.zeros((e_pad - e,), jnp.int32)])
    dstp = jnp.concatenate([edge_index[1], jnp.full((e_pad - e,), N, jnp.int32)])
    asd_flat = jnp.concatenate(
        [asd, jnp.zeros((NP - N, 2 * H), f32)]).reshape(-1)
    xw2 = xw.reshape(N * NC, DH)
    m16 = m[0, :L]

    raw = pl.kernel(
        _sc_edge,
        out_type=jax.ShapeDtypeStruct((NC, NP, W), f32),
        mesh=plsc.VectorSubcoreMesh(core_axis_name="c", subcore_axis_name="s",
                                    num_cores=NC),
        compiler_params=pltpu.CompilerParams(needs_layout_passes=False,
                                             use_tc_tiling_on_sc=False),
        scratch_types=[
            pltpu.VMEM_SHARED((NP, W), f32),
            pltpu.VMEM_SHARED((NP * 2 * H,), f32),
            pltpu.VMEM((2, SCB, CH, DH), f32),
            pltpu.VMEM((2, SCB, CH, W), f32),
            pltpu.VMEM((HC, CH), f32),
            pltpu.VMEM((EC,), jnp.int32),
            pltpu.VMEM((EC,), jnp.int32),
            pltpu.VMEM((2, SCB, CH), jnp.int32),
            pltpu.VMEM((2, EC), jnp.int32),
            pltpu.VMEM((2, SCB, HC, CH), jnp.int32),
            pltpu.VMEM((2, SCB, HC, CH), jnp.int32),
            pltpu.VMEM((2, SCB, HC, CH), f32),
            pltpu.VMEM((2, SCB, HC, CH), f32),
            pltpu.VMEM((L,), f32),
            pltpu.VMEM((1264,), f32),
            pltpu.SemaphoreType.DMA,
            pltpu.SemaphoreType.DMA,
            pltpu.SemaphoreType.DMA,
            pltpu.SemaphoreType.DMA,
            pltpu.SemaphoreType.DMA,
            pltpu.SemaphoreType.DMA,
        ],
    )(srcp, dstp, asd_flat, xw2, m16)

    num = jnp.concatenate([raw[0, :, :DH], raw[1, :, :DH]], axis=1)
    a0 = jnp.concatenate(
        [raw[0, :N, DH:DH + HC], raw[1, :N, DH:DH + HC]], axis=1)
    out = pl.pallas_call(
        _dense_post,
        out_shape=jax.ShapeDtypeStruct((N, D), f32),
    )(num, a0, asd, xw, t, expm, gat_bias.reshape(1, D), m)
    return out


# speedup vs baseline: 55.1156x; 1.0120x over previous
import jax
import jax.numpy as jnp
from jax import lax
from jax.experimental import pallas as pl
from jax.experimental.pallas import tpu as pltpu
from jax.experimental.pallas import tpu_sc as plsc

N = 10000
D = 128
H = 4
C = 32
NEG = 0.2
EPS = 1e-5

NC, NS, L = 2, 16, 16
DH = D // NC
HC = H // NC
W = DH + 8
CH = 128
SCB = 2
EC = SCB * CH
NP = 10112
RPT = NP // NS


def _dense_pre(x_ref, w_ref, a8_ref, fw_ref, fb_ref, g_ref, b_ref,
               xw_ref, asd_ref, t_ref, m_ref):
    x = x_ref[...]
    xw = jnp.dot(x, w_ref[...], preferred_element_type=jnp.float32)
    asd = jnp.dot(xw, a8_ref[...], preferred_element_type=jnp.float32)
    xw_ref[...] = xw
    asd_ref[...] = asd
    mraw = jnp.max(asd[:, :H]) + jnp.max(asd[:, H:])
    m = jnp.maximum(mraw, NEG * mraw)
    m_ref[...] = jnp.broadcast_to(m, (1, D))
    y = jnp.dot(x, fw_ref[...], preferred_element_type=jnp.float32) + fb_ref[...]
    mu = jnp.mean(y, axis=0, keepdims=True)
    var = jnp.mean((y - mu) ** 2, axis=0, keepdims=True)
    t = (y - mu) / jnp.sqrt(var + EPS) * g_ref[...] + b_ref[...]
    t_ref[...] = jnp.maximum(t, 0.0)


def _sc_edge(src_ref, dst_ref, asd_ref, xw2_ref, m_ref, raw_ref,
             msg_sh, asd_sh, rows_v, msg_v, p_v,
             src_v, dst_v, dstx_v, sidx_v, fsb_v, fdb_v, ga_v, gb_v,
             m_v, stage_v, semg0, semg1, seml0, seml1, sems0, sems1):
    cid = lax.axis_index("c")
    sid = lax.axis_index("s")
    nsc = src_ref.shape[0] // NS // EC
    semg = (semg0, semg1)
    seml = (seml0, seml1)
    sems = (sems0, sems1)

    pltpu.sync_copy(m_ref, m_v)

    spt = asd_ref.shape[0] // NS
    s0 = pl.multiple_of(sid * spt, 8)
    for q in range(spt // 1264):
        pltpu.sync_copy(asd_ref.at[pl.ds(s0 + q * 1264, 1264)], stage_v)
        pltpu.sync_copy(stage_v, asd_sh.at[pl.ds(s0 + q * 1264, 1264)])

    zv = jnp.zeros((L,), jnp.float32)

    @pl.loop(0, CH)
    def _zero_msg(k):
        for s in range(2):
            for b in range(SCB):
                for o in (0, L, 2 * L, 3 * L, W - L):
                    msg_v[s, b, k, pl.ds(o, L)] = zv

    r0 = pl.multiple_of(sid * RPT, 8)
    for off, sz in ((0, 128), (128, 128), (256, 128), (384, 128), (512, 120)):
        pltpu.sync_copy(msg_v.at[0, 0, pl.ds(0, sz)],
                        msg_sh.at[pl.ds(r0 + off, sz)])
    plsc.subcore_barrier()

    iota = lax.iota(jnp.int32, L)
    fullk = [jnp.full((L,), k, jnp.int32) for k in range(L)]
    colc = [jnp.full((L,), DH + j, jnp.int32) for j in range(HC)]
    mvec = m_v[...]
    e_base = sid * nsc * EC

    def build_fire(s, t):
        e0 = pl.multiple_of(e_base + t * EC, 8)
        pltpu.sync_copy(src_ref.at[pl.ds(e0, EC)], src_v)
        pltpu.sync_copy(dst_ref.at[pl.ds(e0, EC)], dst_v)
        for b in range(SCB):
            for k0 in range(0, CH, L):
                kk = b * CH + k0
                s16 = src_v[pl.ds(kk, L)]
                d16 = dst_v[pl.ds(kk, L)]
                sidx_v[s, pl.ds(kk, L)] = s16 * 2 + cid
                dstx_v[s, b, pl.ds(k0, L)] = d16
                fsb_v[s, b, 0, pl.ds(k0, L)] = s16 * 8 + cid * HC
                fsb_v[s, b, 1, pl.ds(k0, L)] = s16 * 8 + (cid * HC + 1)
                fdb_v[s, b, 0, pl.ds(k0, L)] = d16 * 8 + (H + cid * HC)
                fdb_v[s, b, 1, pl.ds(k0, L)] = d16 * 8 + (H + cid * HC + 1)
        for b in range(SCB):
            pltpu.async_copy(xw2_ref.at[sidx_v.at[s, pl.ds(b * CH, CH)]],
                             rows_v.at[s, b], semg[s])
            for j in range(HC):
                pltpu.async_copy(asd_sh.at[fsb_v.at[s, b, j]],
                                 ga_v.at[s, b, j], seml[s])
                pltpu.async_copy(asd_sh.at[fdb_v.at[s, b, j]],
                                 gb_v.at[s, b, j], seml[s])

    def wait_gathers(s):
        for b in range(SCB):
            pltpu.make_async_copy(
                xw2_ref.at[sidx_v.at[s, pl.ds(b * CH, CH)]],
                rows_v.at[s, b], semg[s]).wait()
            for j in range(HC):
                pltpu.make_async_copy(asd_sh.at[fsb_v.at[s, b, j]],
                                      ga_v.at[s, b, j], seml[s]).wait()
                pltpu.make_async_copy(asd_sh.at[fdb_v.at[s, b, j]],
                                      gb_v.at[s, b, j], seml[s]).wait()

    def compute_fire(s):
        for b in range(SCB):
            for k0 in range(0, CH, L):
                kvec = iota + k0
                for j in range(HC):
                    al = ga_v[s, b, j, pl.ds(k0, L)] + gb_v[s, b, j, pl.ds(k0, L)]
                    al = jnp.maximum(al, NEG * al)
                    pv = jnp.exp(al - mvec)
                    p_v[j, pl.ds(k0, L)] = pv
                    plsc.store_scatter(msg_v.at[s, b], [kvec, colc[j]], pv)

            @pl.loop(0, CH, step=L)
            def _row(k0):
                for j in range(HC):
                    ph = p_v[j, pl.ds(k0, L)]
                    for k in range(L):
                        cf = ph[k]
                        for i in range(2):
                            sl = pl.ds(j * C + i * L, L)
                            msg_v[s, b, k0 + k, sl] = \
                                rows_v[s, b, k0 + k, sl] * cf

            pltpu.async_copy(msg_v.at[s, b], msg_sh.at[dstx_v.at[s, b]],
                             sems[s], add=True)

    def drain_scatter(s):
        for b in range(SCB):
            pltpu.make_async_copy(msg_v.at[s, b], msg_sh.at[dstx_v.at[s, b]],
                                  sems[s]).wait()

    build_fire(0, 0)

    @pl.loop(0, nsc, step=2)
    def _pair(t):
        build_fire(1, t + 1)
        wait_gathers(0)
        compute_fire(0)
        drain_scatter(0)

        @pl.when(t + 2 < nsc)
        def _pf():
            build_fire(0, t + 2)

        wait_gathers(1)
        compute_fire(1)
        drain_scatter(1)

    plsc.subcore_barrier()
    for off, sz in ((0, 128), (128, 128), (256, 128), (384, 128), (512, 120)):
        pltpu.sync_copy(msg_sh.at[pl.ds(r0 + off, sz)],
                        raw_ref.at[cid, pl.ds(r0 + off, sz)])


def _dense_post(raw_ref, a0_ref, asd_ref, xw_ref, t_ref, exp_ref,
                bias_ref, m_ref, o_ref):
    num = raw_ref[:N, :]
    asum4 = a0_ref[...]
    asd = asd_ref[...]
    aself = asd[:, :H] + asd[:, H:]
    aself = jnp.maximum(aself, NEG * aself)
    es4 = jnp.exp(aself - m_ref[0:1, 0:1])
    expm = exp_ref[...]
    den = jnp.dot(asum4 + es4, expm, preferred_element_type=jnp.float32) + 1e-16
    num = num + jnp.dot(es4, expm, preferred_element_type=jnp.float32) * xw_ref[...]
    o_ref[...] = num / den + bias_ref[...] + t_ref[...]


def kernel(x, edge_index, edge_attr, spatial_coords,
           gat_W, gat_att_src, gat_att_dst, gat_bias,
           sa_w1, sa_b1, sa_w2, sa_b2,
           ft_w, ft_b, bn_gamma, bn_beta):
    f32 = jnp.float32
    expm = jnp.kron(jnp.eye(H, dtype=f32), jnp.ones((1, C), f32))
    a8 = jnp.concatenate(
        [expm.T * gat_att_src.reshape(-1)[:, None],
         expm.T * gat_att_dst.reshape(-1)[:, None]], axis=1)

    xw, asd, t, m = pl.pallas_call(
        _dense_pre,
        out_shape=[
            jax.ShapeDtypeStruct((N, D), f32),
            jax.ShapeDtypeStruct((N, 2 * H), f32),
            jax.ShapeDtypeStruct((N, D), f32),
            jax.ShapeDtypeStruct((1, D), f32),
        ],
    )(x, gat_W, a8, ft_w, ft_b.reshape(1, D), bn_gamma.reshape(1, D),
      bn_beta.reshape(1, D))

    e = edge_index.shape[1]
    nsc = -(-e // (NS * EC))
    nsc = nsc + (nsc % 2)
    e_pad = nsc * NS * EC
    srcp = jnp.concatenate([edge_index[0], jnp.zeros((e_pad - e,), jnp.int32)])
    dstp = jnp.concatenate([edge_index[1], jnp.full((e_pad - e,), N, jnp.int32)])
    asd_flat = jnp.concatenate(
        [asd, jnp.zeros((NP - N, 2 * H), f32)]).reshape(-1)
    xw2 = xw.reshape(N * NC, DH)
    m16 = m[0, :L]

    raw = pl.kernel(
        _sc_edge,
        out_type=jax.ShapeDtypeStruct((NC, NP, W), f32),
        mesh=plsc.VectorSubcoreMesh(core_axis_name="c", subcore_axis_name="s",
                                    num_cores=NC),
        compiler_params=pltpu.CompilerParams(needs_layout_passes=False,
                                             use_tc_tiling_on_sc=False),
        scratch_types=[
            pltpu.VMEM_SHARED((NP, W), f32),
            pltpu.VMEM_SHARED((NP * 2 * H,), f32),
            pltpu.VMEM((2, SCB, CH, DH), f32),
            pltpu.VMEM((2, SCB, CH, W), f32),
            pltpu.VMEM((HC, CH), f32),
            pltpu.VMEM((EC,), jnp.int32),
            pltpu.VMEM((EC,), jnp.int32),
            pltpu.VMEM((2, SCB, CH), jnp.int32),
            pltpu.VMEM((2, EC), jnp.int32),
            pltpu.VMEM((2, SCB, HC, CH), jnp.int32),
            pltpu.VMEM((2, SCB, HC, CH), jnp.int32),
            pltpu.VMEM((2, SCB, HC, CH), f32),
            pltpu.VMEM((2, SCB, HC, CH), f32),
            pltpu.VMEM((L,), f32),
            pltpu.VMEM((1264,), f32),
            pltpu.SemaphoreType.DMA,
            pltpu.SemaphoreType.DMA,
            pltpu.SemaphoreType.DMA,
            pltpu.SemaphoreType.DMA,
            pltpu.SemaphoreType.DMA,
            pltpu.SemaphoreType.DMA,
        ],
    )(srcp, dstp, asd_flat, xw2, m16)

    num = jnp.concatenate([raw[0, :, :DH], raw[1, :, :DH]], axis=1)
    a0 = jnp.concatenate(
        [raw[0, :N, DH:DH + HC], raw[1, :N, DH:DH + HC]], axis=1)
    out = pl.pallas_call(
        _dense_post,
        out_shape=jax.ShapeDtypeStruct((N, D), f32),
    )(num, a0, asd, xw, t, expm, gat_bias.reshape(1, D), m)
    return out


# speedup vs baseline: 62.2883x; 1.1301x over previous
import jax
import jax.numpy as jnp
from jax import lax
from jax.experimental import pallas as pl
from jax.experimental.pallas import tpu as pltpu
from jax.experimental.pallas import tpu_sc as plsc

N = 10000
D = 128
H = 4
C = 32
NEG = 0.2
EPS = 1e-5

NC, NS, L = 2, 16, 16
DH = D // NC
HC = H // NC
W = DH + 8
CH = 128
SCB = 2
EC = SCB * CH
NP = 10112
RPT = NP // NS


def _dense_pre(x_ref, w_ref, a8_ref, fw_ref, fb_ref, g_ref, b_ref,
               xw_ref, asd_ref, t_ref, m_ref):
    x = x_ref[...]
    xw = jnp.dot(x, w_ref[...], preferred_element_type=jnp.float32)
    asd = jnp.dot(xw, a8_ref[...], preferred_element_type=jnp.float32)
    xw_ref[...] = xw
    asd_ref[...] = asd
    mraw = jnp.max(asd[:, :H]) + jnp.max(asd[:, H:])
    m = jnp.maximum(mraw, NEG * mraw)
    m_ref[...] = jnp.broadcast_to(m, (1, D))
    y = jnp.dot(x, fw_ref[...], preferred_element_type=jnp.float32) + fb_ref[...]
    mu = jnp.mean(y, axis=0, keepdims=True)
    var = jnp.mean((y - mu) ** 2, axis=0, keepdims=True)
    t = (y - mu) / jnp.sqrt(var + EPS) * g_ref[...] + b_ref[...]
    t_ref[...] = jnp.maximum(t, 0.0)


def _sc_edge(src_ref, dst_ref, asd_ref, xw2_ref, m_ref, raw_ref,
             msg_sh, asd_sh, rows_v, msg_v, p_v,
             src_v, dst_v, dstx_v, sidx_v, fsb_v, fdb_v, ga_v, gb_v,
             m_v, stage_v, semg0, semg1, seml0, seml1, sems0, sems1):
    cid = lax.axis_index("c")
    sid = lax.axis_index("s")
    nsc = src_ref.shape[0] // NS // EC
    semg = (semg0, semg1)
    seml = (seml0, seml1)
    sems = (sems0, sems1)

    pltpu.sync_copy(m_ref, m_v)

    spt = asd_ref.shape[0] // NS
    s0 = pl.multiple_of(sid * spt, 8)
    for q in range(spt // 1264):
        pltpu.sync_copy(asd_ref.at[pl.ds(s0 + q * 1264, 1264)], stage_v)
        pltpu.sync_copy(stage_v, asd_sh.at[pl.ds(s0 + q * 1264, 1264)])

    zv = jnp.zeros((L,), jnp.float32)

    @pl.loop(0, CH)
    def _zero_msg(k):
        for s in range(2):
            for b in range(SCB):
                for o in (0, L, 2 * L, 3 * L, W - L):
                    msg_v[s, b, k, pl.ds(o, L)] = zv

    r0 = pl.multiple_of(sid * RPT, 8)
    for off, sz in ((0, 128), (128, 128), (256, 128), (384, 128), (512, 120)):
        pltpu.sync_copy(msg_v.at[0, 0, pl.ds(0, sz)],
                        msg_sh.at[pl.ds(r0 + off, sz)])
    plsc.subcore_barrier()

    iota = lax.iota(jnp.int32, L)
    fullk = [jnp.full((L,), k, jnp.int32) for k in range(L)]
    colc = [jnp.full((L,), DH + j, jnp.int32) for j in range(HC)]
    mvec = m_v[...]
    e_base = sid * nsc * EC

    def build_fire(s, t):
        e0 = pl.multiple_of(e_base + t * EC, 8)
        pltpu.sync_copy(src_ref.at[pl.ds(e0, EC)], src_v)
        pltpu.sync_copy(dst_ref.at[pl.ds(e0, EC)], dst_v)
        for b in range(SCB):
            for k0 in range(0, CH, L):
                kk = b * CH + k0
                s16 = src_v[pl.ds(kk, L)]
                d16 = dst_v[pl.ds(kk, L)]
                sidx_v[s, pl.ds(kk, L)] = s16 * 2 + cid
                dstx_v[s, b, pl.ds(k0, L)] = d16
                fsb_v[s, b, 0, pl.ds(k0, L)] = s16 * 8 + cid * HC
                fsb_v[s, b, 1, pl.ds(k0, L)] = s16 * 8 + (cid * HC + 1)
                fdb_v[s, b, 0, pl.ds(k0, L)] = d16 * 8 + (H + cid * HC)
                fdb_v[s, b, 1, pl.ds(k0, L)] = d16 * 8 + (H + cid * HC + 1)
        for b in range(SCB):
            pltpu.async_copy(xw2_ref.at[sidx_v.at[s, pl.ds(b * CH, CH)]],
                             rows_v.at[s, b], semg[s])
            for j in range(HC):
                pltpu.async_copy(asd_sh.at[fsb_v.at[s, b, j]],
                                 ga_v.at[s, b, j], seml[s])
                pltpu.async_copy(asd_sh.at[fdb_v.at[s, b, j]],
                                 gb_v.at[s, b, j], seml[s])

    def wait_gathers(s):
        for b in range(SCB):
            pltpu.make_async_copy(
                xw2_ref.at[sidx_v.at[s, pl.ds(b * CH, CH)]],
                rows_v.at[s, b], semg[s]).wait()
            for j in range(HC):
                pltpu.make_async_copy(asd_sh.at[fsb_v.at[s, b, j]],
                                      ga_v.at[s, b, j], seml[s]).wait()
                pltpu.make_async_copy(asd_sh.at[fdb_v.at[s, b, j]],
                                      gb_v.at[s, b, j], seml[s]).wait()

    def compute_fire(s):
        for b in range(SCB):
            for k0 in range(0, CH, L):
                kvec = iota + k0
                for j in range(HC):
                    al = ga_v[s, b, j, pl.ds(k0, L)] + gb_v[s, b, j, pl.ds(k0, L)]
                    al = jnp.maximum(al, NEG * al)
                    pv = jnp.exp(al - mvec)
                    p_v[j, pl.ds(k0, L)] = pv
                    plsc.store_scatter(msg_v.at[s, b], [kvec, colc[j]], pv)

            @plsc.parallel_loop(0, CH, step=L, unroll=2)
            def _row(k0):
                for j in range(HC):
                    ph = p_v[j, pl.ds(k0, L)]
                    for k in range(L):
                        cf = ph[k]
                        for i in range(2):
                            sl = pl.ds(j * C + i * L, L)
                            msg_v[s, b, k0 + k, sl] = \
                                rows_v[s, b, k0 + k, sl] * cf

            pltpu.async_copy(msg_v.at[s, b], msg_sh.at[dstx_v.at[s, b]],
                             sems[s], add=True)

    def drain_scatter(s):
        for b in range(SCB):
            pltpu.make_async_copy(msg_v.at[s, b], msg_sh.at[dstx_v.at[s, b]],
                                  sems[s]).wait()

    build_fire(0, 0)

    @pl.loop(0, nsc, step=2)
    def _pair(t):
        build_fire(1, t + 1)
        wait_gathers(0)
        compute_fire(0)
        drain_scatter(0)

        @pl.when(t + 2 < nsc)
        def _pf():
            build_fire(0, t + 2)

        wait_gathers(1)
        compute_fire(1)
        drain_scatter(1)

    plsc.subcore_barrier()
    for off, sz in ((0, 128), (128, 128), (256, 128), (384, 128), (512, 120)):
        pltpu.sync_copy(msg_sh.at[pl.ds(r0 + off, sz)],
                        raw_ref.at[cid, pl.ds(r0 + off, sz)])


def _dense_post(raw_ref, a0_ref, asd_ref, xw_ref, t_ref, exp_ref,
                bias_ref, m_ref, o_ref):
    num = raw_ref[:N, :]
    asum4 = a0_ref[...]
    asd = asd_ref[...]
    aself = asd[:, :H] + asd[:, H:]
    aself = jnp.maximum(aself, NEG * aself)
    es4 = jnp.exp(aself - m_ref[0:1, 0:1])
    expm = exp_ref[...]
    den = jnp.dot(asum4 + es4, expm, preferred_element_type=jnp.float32) + 1e-16
    num = num + jnp.dot(es4, expm, preferred_element_type=jnp.float32) * xw_ref[...]
    o_ref[...] = num / den + bias_ref[...] + t_ref[...]


def kernel(x, edge_index, edge_attr, spatial_coords,
           gat_W, gat_att_src, gat_att_dst, gat_bias,
           sa_w1, sa_b1, sa_w2, sa_b2,
           ft_w, ft_b, bn_gamma, bn_beta):
    f32 = jnp.float32
    expm = jnp.kron(jnp.eye(H, dtype=f32), jnp.ones((1, C), f32))
    a8 = jnp.concatenate(
        [expm.T * gat_att_src.reshape(-1)[:, None],
         expm.T * gat_att_dst.reshape(-1)[:, None]], axis=1)

    xw, asd, t, m = pl.pallas_call(
        _dense_pre,
        out_shape=[
            jax.ShapeDtypeStruct((N, D), f32),
            jax.ShapeDtypeStruct((N, 2 * H), f32),
            jax.ShapeDtypeStruct((N, D), f32),
            jax.ShapeDtypeStruct((1, D), f32),
        ],
    )(x, gat_W, a8, ft_w, ft_b.reshape(1, D), bn_gamma.reshape(1, D),
      bn_beta.reshape(1, D))

    e = edge_index.shape[1]
    nsc = -(-e // (NS * EC))
    nsc = nsc + (nsc % 2)
    e_pad = nsc * NS * EC
    srcp = jnp.concatenate([edge_index[0], jnp.zeros((e_pad - e,), jnp.int32)])
    dstp = jnp.concatenate([edge_index[1], jnp.full((e_pad - e,), N, jnp.int32)])
    asd_flat = jnp.concatenate(
        [asd, jnp.zeros((NP - N, 2 * H), f32)]).reshape(-1)
    xw2 = xw.reshape(N * NC, DH)
    m16 = m[0, :L]

    raw = pl.kernel(
        _sc_edge,
        out_type=jax.ShapeDtypeStruct((NC, NP, W), f32),
        mesh=plsc.VectorSubcoreMesh(core_axis_name="c", subcore_axis_name="s",
                                    num_cores=NC),
        compiler_params=pltpu.CompilerParams(needs_layout_passes=False,
                                             use_tc_tiling_on_sc=False),
        scratch_types=[
            pltpu.VMEM_SHARED((NP, W), f32),
            pltpu.VMEM_SHARED((NP * 2 * H,), f32),
            pltpu.VMEM((2, SCB, CH, DH), f32),
            pltpu.VMEM((2, SCB, CH, W), f32),
            pltpu.VMEM((HC, CH), f32),
            pltpu.VMEM((EC,), jnp.int32),
            pltpu.VMEM((EC,), jnp.int32),
            pltpu.VMEM((2, SCB, CH), jnp.int32),
            pltpu.VMEM((2, EC), jnp.int32),
            pltpu.VMEM((2, SCB, HC, CH), jnp.int32),
            pltpu.VMEM((2, SCB, HC, CH), jnp.int32),
            pltpu.VMEM((2, SCB, HC, CH), f32),
            pltpu.VMEM((2, SCB, HC, CH), f32),
            pltpu.VMEM((L,), f32),
            pltpu.VMEM((1264,), f32),
            pltpu.SemaphoreType.DMA,
            pltpu.SemaphoreType.DMA,
            pltpu.SemaphoreType.DMA,
            pltpu.SemaphoreType.DMA,
            pltpu.SemaphoreType.DMA,
            pltpu.SemaphoreType.DMA,
        ],
    )(srcp, dstp, asd_flat, xw2, m16)

    num = jnp.concatenate([raw[0, :, :DH], raw[1, :, :DH]], axis=1)
    a0 = jnp.concatenate(
        [raw[0, :N, DH:DH + HC], raw[1, :N, DH:DH + HC]], axis=1)
    out = pl.pallas_call(
        _dense_post,
        out_shape=jax.ShapeDtypeStruct((N, D), f32),
    )(num, a0, asd, xw, t, expm, gat_bias.reshape(1, D), m)
    return out
